# baseline (speedup 1.0000x reference)
import functools
import math

import numpy as np
import jax
import jax.numpy as jnp
from jax import lax
from jax.experimental import pallas as pl
from jax.experimental.pallas import tpu as pltpu
from jax.experimental.pallas import tpu_sc as plsc

N_SPHERICAL = 7
N_RADIAL = 6
CUTOFF = 5.0
ENV_EXPONENT = 5


def _sph_jn_scalar(l, x):
    j0 = math.sin(x) / x
    if l == 0:
        return j0
    jm1 = j0
    j = math.sin(x) / x ** 2 - math.cos(x) / x
    for i in range(2, l + 1):
        jm1, j = j, (2 * i - 1) / x * j - jm1
    return j


def _jn_zeros(n, k):
    zerosj = np.zeros((n, k))
    zerosj[0] = np.arange(1, k + 1) * np.pi
    points = np.arange(1, k + n) * np.pi
    racines = np.zeros(k + n - 1)
    for i in range(1, n):
        for j in range(k + n - 1 - i):
            a = points[j]
            b = points[j + 1]
            fa = _sph_jn_scalar(i, a)
            for _ in range(200):
                m = 0.5 * (a + b)
                fm = _sph_jn_scalar(i, m)
                if fa * fm <= 0.0:
                    b = m
                else:
                    a = m
                    fa = fm
            racines[j] = 0.5 * (a + b)
        points = racines.copy()
        zerosj[i, :k] = racines[:k]
    return zerosj


_ZEROS = _jn_zeros(N_SPHERICAL, N_RADIAL)
_NORM = np.zeros((N_SPHERICAL, N_RADIAL))
for _l in range(N_SPHERICAL):
    for _i in range(N_RADIAL):
        _NORM[_l, _i] = 1.0 / math.sqrt(
            0.5 * _sph_jn_scalar(_l + 1, _ZEROS[_l, _i]) ** 2)

_NC = 42
_Z_FLAT = _ZEROS.reshape(_NC).astype(np.float32)
_L_OF_COL = np.repeat(np.arange(N_SPHERICAL), N_RADIAL).astype(np.int32)
_N_FLAT = _NORM.reshape(_NC).astype(np.float32)
_PREF = np.array(
    [math.sqrt((2 * l + 1) / (4.0 * math.pi)) for l in range(N_SPHERICAL)],
    dtype=np.float32)
_PREF_COL = _PREF[_L_OF_COL]


def _sc_gather(dist, idx):
    t = idx.shape[0]
    nc, ns = 2, 16
    nw = nc * ns
    bpw = t // nw
    assert bpw * nw == t and bpw % 8 == 0
    mesh = plsc.VectorSubcoreMesh(core_axis_name="c", subcore_axis_name="s")

    @functools.partial(
        pl.kernel,
        mesh=mesh,
        out_type=jax.ShapeDtypeStruct((t,), jnp.float32),
        scratch_types=[
            pltpu.VMEM((bpw,), jnp.int32),
            pltpu.VMEM((bpw,), jnp.float32),
            pltpu.SemaphoreType.DMA,
        ],
    )
    def gk(dist_hbm, idx_hbm, out_hbm, idx_v, val_v, sem):
        wid = lax.axis_index("s") * nc + lax.axis_index("c")
        base = wid * bpw
        pltpu.sync_copy(idx_hbm.at[pl.ds(base, bpw)], idx_v)
        pltpu.async_copy(dist_hbm.at[idx_v], val_v, sem).wait()
        pltpu.sync_copy(val_v, out_hbm.at[pl.ds(base, bpw)])

    return gk(dist, idx)


_PIO2_1 = 1.5707855224609375
_PIO2_1T = 1.0804334124e-05
_TWO_OPI = 0.6366197723675814


def _fast_sincos(a):
    nf = jnp.floor(a * _TWO_OPI + 0.5)
    r = (a - nf * _PIO2_1) - nf * _PIO2_1T
    r2 = r * r
    sp = r + (r2 * r) * (-1.6666667163e-01 + r2 * (
        8.3333337680e-03 + r2 * (-1.9841270114e-04 + r2 * 2.7557314297e-06)))
    r4 = r2 * r2
    cp = 1.0 - 0.5 * r2 + r4 * (4.1666667908e-02 + r2 * (
        -1.3888889225e-03 + r2 * 2.4801587642e-05))
    ni = nf.astype(jnp.int32)
    b0 = (ni & 1) == 1
    b1 = (ni & 2) == 2
    b1c = ((ni + 1) & 2) == 2
    s = jnp.where(b0, cp, sp)
    s = jnp.where(b1, -s, s)
    c = jnp.where(b0, sp, cp)
    c = jnp.where(b1c, -c, c)
    return s, c


_DFACT = np.array([np.prod(np.arange(1, 2 * l + 2, 2.0))
                   for l in range(N_SPHERICAL)])
_LS = np.arange(N_SPHERICAL).astype(np.float64)
_SER = np.stack([
    0.6 * (2 * _LS + 3),
    1.0 / _DFACT,
    -1.0 / (2 * (2 * _LS + 3)),
    1.0 / (8 * (2 * _LS + 3) * (2 * _LS + 5)),
    -1.0 / (48 * (2 * _LS + 3) * (2 * _LS + 5) * (2 * _LS + 7)),
], axis=1).astype(np.float32)[_L_OF_COL]


def _tc_body(zs_ref, cn_ref, lcol_ref, ser_ref, dist_ref, ang_ref, out_ref):
    zs = zs_ref[...]
    cn = cn_ref[...]
    lcol = lcol_ref[...]
    ath2 = ser_ref[:, 0:1]
    sc0 = ser_ref[:, 1:2]
    sc1 = ser_ref[:, 2:3]
    sc2 = ser_ref[:, 3:4]
    sc3 = ser_ref[:, 4:5]

    d = dist_ref[...].reshape(1, -1)
    x = d * (1.0 / CUTOFF)
    a = zs * x
    s, c = _fast_sincos(a)
    inv = 1.0 / a
    j0 = s * inv
    j1 = (j0 - c) * inv
    res = jnp.where(lcol == 0, j0, j1)
    jm1, jcur = j0, j1
    for i in range(2, N_SPHERICAL):
        jm1, jcur = jcur, (2 * i - 1) * inv * jcur - jm1
        res = jnp.where(lcol == i, jcur, res)

    t = a * a
    al = jnp.where(lcol == 0, jnp.ones_like(a), a)
    p = a
    for i in range(2, N_SPHERICAL):
        p = p * a
        al = jnp.where(lcol == i, p, al)
    ser = (al * sc0) * (1.0 + t * (sc1 + t * (sc2 + t * sc3)))
    res = jnp.where(t < ath2, ser, res)

    x2 = x * x
    x4 = x2 * x2
    x5 = x * x4
    env = 1.0 / x + x5 * (-28.0 + x * (48.0 + x * (-21.0)))
    env = jnp.where(x < 1.0, env, jnp.zeros_like(env))
    res = (cn * res) * env

    ct = _fast_sincos(ang_ref[...].reshape(1, -1))[1]
    psel = jnp.where(lcol == 0, jnp.ones_like(ct), ct)
    pm1, pc = jnp.ones_like(ct), ct
    for l in range(2, N_SPHERICAL):
        pm1, pc = pc, ((2 * l - 1) * ct * pc - (l - 1) * pm1) * (1.0 / l)
        psel = jnp.where(lcol == l, pc, psel)
    out_ref[...] = res * psel


def _tc_compute(dist_kj, angle, block_t=8192, interpret=False):
    t = dist_kj.shape[0]
    nb = (t + block_t - 1) // block_t
    assert block_t % 1024 == 0
    const_spec = pl.BlockSpec((_NC, 1), lambda i: (0, 0))
    return pl.pallas_call(
        _tc_body,
        grid=(nb,),
        in_specs=[
            const_spec,
            const_spec,
            const_spec,
            pl.BlockSpec((_NC, 5), lambda i: (0, 0)),
            pl.BlockSpec((block_t,), lambda i: (i,)),
            pl.BlockSpec((block_t,), lambda i: (i,)),
        ],
        out_specs=pl.BlockSpec((_NC, block_t), lambda i: (0, i)),
        out_shape=jax.ShapeDtypeStruct((_NC, t), jnp.float32),
        interpret=interpret,
    )(
        jnp.asarray(_Z_FLAT)[:, None],
        jnp.asarray(_N_FLAT * _PREF_COL)[:, None],
        jnp.asarray(_L_OF_COL)[:, None],
        jnp.asarray(_SER),
        dist_kj,
        angle,
    )


def kernel(dist, angle, edge_idx_kj):
    dist_kj = _sc_gather(dist, edge_idx_kj)
    out_t = _tc_compute(dist_kj, angle)
    return jnp.transpose(out_t, (1, 0))


# speedup vs baseline: 1.9300x; 1.9300x over previous
import functools
import math

import numpy as np
import jax
import jax.numpy as jnp
from jax import lax
from jax.experimental import pallas as pl
from jax.experimental.pallas import tpu as pltpu
from jax.experimental.pallas import tpu_sc as plsc

N_SPHERICAL = 7
N_RADIAL = 6
CUTOFF = 5.0
ENV_EXPONENT = 5


def _sph_jn_scalar(l, x):
    j0 = math.sin(x) / x
    if l == 0:
        return j0
    jm1 = j0
    j = math.sin(x) / x ** 2 - math.cos(x) / x
    for i in range(2, l + 1):
        jm1, j = j, (2 * i - 1) / x * j - jm1
    return j


def _jn_zeros(n, k):
    zerosj = np.zeros((n, k))
    zerosj[0] = np.arange(1, k + 1) * np.pi
    points = np.arange(1, k + n) * np.pi
    racines = np.zeros(k + n - 1)
    for i in range(1, n):
        for j in range(k + n - 1 - i):
            a = points[j]
            b = points[j + 1]
            fa = _sph_jn_scalar(i, a)
            for _ in range(200):
                m = 0.5 * (a + b)
                fm = _sph_jn_scalar(i, m)
                if fa * fm <= 0.0:
                    b = m
                else:
                    a = m
                    fa = fm
            racines[j] = 0.5 * (a + b)
        points = racines.copy()
        zerosj[i, :k] = racines[:k]
    return zerosj


_ZEROS = _jn_zeros(N_SPHERICAL, N_RADIAL)
_NORM = np.zeros((N_SPHERICAL, N_RADIAL))
for _l in range(N_SPHERICAL):
    for _i in range(N_RADIAL):
        _NORM[_l, _i] = 1.0 / math.sqrt(
            0.5 * _sph_jn_scalar(_l + 1, _ZEROS[_l, _i]) ** 2)

_NC = 42
_Z_FLAT = _ZEROS.reshape(_NC).astype(np.float32)
_L_OF_COL = np.repeat(np.arange(N_SPHERICAL), N_RADIAL).astype(np.int32)
_N_FLAT = _NORM.reshape(_NC).astype(np.float32)
_PREF = np.array(
    [math.sqrt((2 * l + 1) / (4.0 * math.pi)) for l in range(N_SPHERICAL)],
    dtype=np.float32)
_PREF_COL = _PREF[_L_OF_COL]


def _sc_gather(dist, idx):
    t = idx.shape[0]
    nc, ns = 2, 16
    nw = nc * ns
    bpw = t // nw
    assert bpw * nw == t and bpw % 8 == 0
    mesh = plsc.VectorSubcoreMesh(core_axis_name="c", subcore_axis_name="s")

    @functools.partial(
        pl.kernel,
        mesh=mesh,
        out_type=jax.ShapeDtypeStruct((t,), jnp.float32),
        scratch_types=[
            pltpu.VMEM((bpw,), jnp.int32),
            pltpu.VMEM((bpw,), jnp.float32),
            pltpu.SemaphoreType.DMA,
        ],
    )
    def gk(dist_hbm, idx_hbm, out_hbm, idx_v, val_v, sem):
        wid = lax.axis_index("s") * nc + lax.axis_index("c")
        base = wid * bpw
        pltpu.sync_copy(idx_hbm.at[pl.ds(base, bpw)], idx_v)
        pltpu.async_copy(dist_hbm.at[idx_v], val_v, sem).wait()
        pltpu.sync_copy(val_v, out_hbm.at[pl.ds(base, bpw)])

    return gk(dist, idx)


_PIO2_1 = 1.5707855224609375
_PIO2_1T = 1.0804334124e-05
_TWO_OPI = 0.6366197723675814


def _fast_sincos(a):
    nf = jnp.floor(a * _TWO_OPI + 0.5)
    r = (a - nf * _PIO2_1) - nf * _PIO2_1T
    r2 = r * r
    sp = r + (r2 * r) * (-1.6666667163e-01 + r2 * (
        8.3333337680e-03 + r2 * (-1.9841270114e-04 + r2 * 2.7557314297e-06)))
    r4 = r2 * r2
    cp = 1.0 - 0.5 * r2 + r4 * (4.1666667908e-02 + r2 * (
        -1.3888889225e-03 + r2 * 2.4801587642e-05))
    ni = nf.astype(jnp.int32)
    b0 = (ni & 1) == 1
    b1 = (ni & 2) == 2
    b1c = ((ni + 1) & 2) == 2
    s = jnp.where(b0, cp, sp)
    s = jnp.where(b1, -s, s)
    c = jnp.where(b0, sp, cp)
    c = jnp.where(b1c, -c, c)
    return s, c


_DFACT = np.array([np.prod(np.arange(1, 2 * l + 2, 2.0))
                   for l in range(N_SPHERICAL)])
_LS = np.arange(N_SPHERICAL).astype(np.float64)
_SER = np.stack([
    0.6 * (2 * _LS + 3),
    1.0 / _DFACT,
    -1.0 / (2 * (2 * _LS + 3)),
    1.0 / (8 * (2 * _LS + 3) * (2 * _LS + 5)),
    -1.0 / (48 * (2 * _LS + 3) * (2 * _LS + 5) * (2 * _LS + 7)),
], axis=1).astype(np.float32)[_L_OF_COL]


_CHUNK = 256


def _tc_body(zs_ref, cn_ref, lcol_ref, ser_ref, dist_ref, ang_ref, out_ref):
    zs = zs_ref[...]
    cn = cn_ref[...]
    lcol = lcol_ref[...]
    ath2 = ser_ref[:, 0:1]
    sc0 = ser_ref[:, 1:2]
    sc1 = ser_ref[:, 2:3]
    sc2 = ser_ref[:, 3:4]
    sc3 = ser_ref[:, 4:5]
    for ch in range(out_ref.shape[1] // _CHUNK):
        _tc_chunk(zs, cn, lcol, ath2, sc0, sc1, sc2, sc3,
                  dist_ref, ang_ref, out_ref, ch)


def _tc_chunk(zs, cn, lcol, ath2, sc0, sc1, sc2, sc3,
              dist_ref, ang_ref, out_ref, ch):
    d = dist_ref[pl.ds(ch * _CHUNK, _CHUNK)].reshape(1, -1)
    x = d * (1.0 / CUTOFF)
    a = zs * x
    s, c = _fast_sincos(a)
    inv = 1.0 / a
    j0 = s * inv
    j1 = (j0 - c) * inv
    res = jnp.where(lcol == 0, j0, j1)
    jm1, jcur = j0, j1
    for i in range(2, N_SPHERICAL):
        jm1, jcur = jcur, (2 * i - 1) * inv * jcur - jm1
        res = jnp.where(lcol == i, jcur, res)

    t = a * a
    al = jnp.where(lcol == 0, jnp.ones_like(a), a)
    p = a
    for i in range(2, N_SPHERICAL):
        p = p * a
        al = jnp.where(lcol == i, p, al)
    ser = (al * sc0) * (1.0 + t * (sc1 + t * (sc2 + t * sc3)))
    res = jnp.where(t < ath2, ser, res)

    x2 = x * x
    x4 = x2 * x2
    x5 = x * x4
    env = 1.0 / x + x5 * (-28.0 + x * (48.0 + x * (-21.0)))
    env = jnp.where(x < 1.0, env, jnp.zeros_like(env))
    res = (cn * res) * env

    ct = _fast_sincos(ang_ref[pl.ds(ch * _CHUNK, _CHUNK)].reshape(1, -1))[1]
    psel = jnp.where(lcol == 0, jnp.ones_like(ct), ct)
    pm1, pc = jnp.ones_like(ct), ct
    for l in range(2, N_SPHERICAL):
        pm1, pc = pc, ((2 * l - 1) * ct * pc - (l - 1) * pm1) * (1.0 / l)
        psel = jnp.where(lcol == l, pc, psel)
    out_ref[:, pl.ds(ch * _CHUNK, _CHUNK)] = res * psel


def _tc_compute(dist_kj, angle, block_t=8192, interpret=False):
    t = dist_kj.shape[0]
    nb = (t + block_t - 1) // block_t
    assert block_t % 1024 == 0
    const_spec = pl.BlockSpec((_NC, 1), lambda i: (0, 0))
    return pl.pallas_call(
        _tc_body,
        grid=(nb,),
        in_specs=[
            const_spec,
            const_spec,
            const_spec,
            pl.BlockSpec((_NC, 5), lambda i: (0, 0)),
            pl.BlockSpec((block_t,), lambda i: (i,)),
            pl.BlockSpec((block_t,), lambda i: (i,)),
        ],
        out_specs=pl.BlockSpec((_NC, block_t), lambda i: (0, i)),
        out_shape=jax.ShapeDtypeStruct((_NC, t), jnp.float32),
        interpret=interpret,
    )(
        jnp.asarray(_Z_FLAT)[:, None],
        jnp.asarray(_N_FLAT * _PREF_COL)[:, None],
        jnp.asarray(_L_OF_COL)[:, None],
        jnp.asarray(_SER),
        dist_kj,
        angle,
    )


def kernel(dist, angle, edge_idx_kj):
    dist_kj = _sc_gather(dist, edge_idx_kj)
    out_t = _tc_compute(dist_kj, angle)
    return jnp.transpose(out_t, (1, 0))


# speedup vs baseline: 1.9717x; 1.0216x over previous
import functools
import math

import numpy as np
import jax
import jax.numpy as jnp
from jax import lax
from jax.experimental import pallas as pl
from jax.experimental.pallas import tpu as pltpu
from jax.experimental.pallas import tpu_sc as plsc

N_SPHERICAL = 7
N_RADIAL = 6
CUTOFF = 5.0
ENV_EXPONENT = 5


def _sph_jn_scalar(l, x):
    j0 = math.sin(x) / x
    if l == 0:
        return j0
    jm1 = j0
    j = math.sin(x) / x ** 2 - math.cos(x) / x
    for i in range(2, l + 1):
        jm1, j = j, (2 * i - 1) / x * j - jm1
    return j


def _jn_zeros(n, k):
    zerosj = np.zeros((n, k))
    zerosj[0] = np.arange(1, k + 1) * np.pi
    points = np.arange(1, k + n) * np.pi
    racines = np.zeros(k + n - 1)
    for i in range(1, n):
        for j in range(k + n - 1 - i):
            a = points[j]
            b = points[j + 1]
            fa = _sph_jn_scalar(i, a)
            for _ in range(200):
                m = 0.5 * (a + b)
                fm = _sph_jn_scalar(i, m)
                if fa * fm <= 0.0:
                    b = m
                else:
                    a = m
                    fa = fm
            racines[j] = 0.5 * (a + b)
        points = racines.copy()
        zerosj[i, :k] = racines[:k]
    return zerosj


_ZEROS = _jn_zeros(N_SPHERICAL, N_RADIAL)
_NORM = np.zeros((N_SPHERICAL, N_RADIAL))
for _l in range(N_SPHERICAL):
    for _i in range(N_RADIAL):
        _NORM[_l, _i] = 1.0 / math.sqrt(
            0.5 * _sph_jn_scalar(_l + 1, _ZEROS[_l, _i]) ** 2)

_NC = 42
_Z_FLAT = _ZEROS.reshape(_NC).astype(np.float32)
_L_OF_COL = np.repeat(np.arange(N_SPHERICAL), N_RADIAL).astype(np.int32)
_N_FLAT = _NORM.reshape(_NC).astype(np.float32)
_PREF = np.array(
    [math.sqrt((2 * l + 1) / (4.0 * math.pi)) for l in range(N_SPHERICAL)],
    dtype=np.float32)
_PREF_COL = _PREF[_L_OF_COL]


def _sc_gather(dist, idx):
    t = idx.shape[0]
    nc, ns = 2, 16
    nw = nc * ns
    bpw = t // nw
    assert bpw * nw == t and bpw % 8 == 0
    mesh = plsc.VectorSubcoreMesh(core_axis_name="c", subcore_axis_name="s")

    @functools.partial(
        pl.kernel,
        mesh=mesh,
        out_type=jax.ShapeDtypeStruct((t,), jnp.float32),
        scratch_types=[
            pltpu.VMEM((bpw,), jnp.int32),
            pltpu.VMEM((bpw,), jnp.float32),
            pltpu.SemaphoreType.DMA,
        ],
    )
    def gk(dist_hbm, idx_hbm, out_hbm, idx_v, val_v, sem):
        wid = lax.axis_index("s") * nc + lax.axis_index("c")
        base = wid * bpw
        pltpu.sync_copy(idx_hbm.at[pl.ds(base, bpw)], idx_v)
        pltpu.async_copy(dist_hbm.at[idx_v], val_v, sem).wait()
        pltpu.sync_copy(val_v, out_hbm.at[pl.ds(base, bpw)])

    return gk(dist, idx)


_PIO2_1 = 1.5707855224609375
_PIO2_1T = 1.0804334124e-05
_TWO_OPI = 0.6366197723675814


def _fast_sincos(a):
    nf = jnp.floor(a * _TWO_OPI + 0.5)
    r = (a - nf * _PIO2_1) - nf * _PIO2_1T
    r2 = r * r
    sp = r + (r2 * r) * (-1.6666667163e-01 + r2 * (
        8.3333337680e-03 + r2 * (-1.9841270114e-04 + r2 * 2.7557314297e-06)))
    r4 = r2 * r2
    cp = 1.0 - 0.5 * r2 + r4 * (4.1666667908e-02 + r2 * (
        -1.3888889225e-03 + r2 * 2.4801587642e-05))
    ni = nf.astype(jnp.int32)
    b0 = (ni & 1) == 1
    b1 = (ni & 2) == 2
    b1c = ((ni + 1) & 2) == 2
    s = jnp.where(b0, cp, sp)
    s = jnp.where(b1, -s, s)
    c = jnp.where(b0, sp, cp)
    c = jnp.where(b1c, -c, c)
    return s, c


_DFACT = np.array([np.prod(np.arange(1, 2 * l + 2, 2.0))
                   for l in range(N_SPHERICAL)])
_LS = np.arange(N_SPHERICAL).astype(np.float64)
_SER = np.stack([
    0.6 * (2 * _LS + 3),
    1.0 / _DFACT,
    -1.0 / (2 * (2 * _LS + 3)),
    1.0 / (8 * (2 * _LS + 3) * (2 * _LS + 5)),
    -1.0 / (48 * (2 * _LS + 3) * (2 * _LS + 5) * (2 * _LS + 7)),
], axis=1).astype(np.float32)[_L_OF_COL]


_CHUNK = 256


def _tc_body(zs_ref, cn_ref, lcol_ref, ser_ref, dist_ref, ang_ref, out_ref):
    zs = zs_ref[...]
    cn = cn_ref[...]
    lcol = lcol_ref[...]
    ath2 = ser_ref[:, 0:1]
    sc0 = ser_ref[:, 1:2]
    sc1 = ser_ref[:, 2:3]
    sc2 = ser_ref[:, 3:4]
    sc3 = ser_ref[:, 4:5]
    masks = [lcol == i for i in range(N_SPHERICAL)]
    for ch in range(out_ref.shape[1] // _CHUNK):
        _tc_chunk(zs, cn, masks, ath2, sc0, sc1, sc2, sc3,
                  dist_ref, ang_ref, out_ref, ch)


def _tc_chunk(zs, cn, masks, ath2, sc0, sc1, sc2, sc3,
              dist_ref, ang_ref, out_ref, ch):
    d = dist_ref[pl.ds(ch * _CHUNK, _CHUNK)].reshape(1, -1)
    x = d * (1.0 / CUTOFF)
    a = zs * x

    t = a * a
    al = jnp.where(masks[0], jnp.ones_like(a), a)
    p = a
    for i in range(2, N_SPHERICAL):
        p = p * a
        al = jnp.where(masks[i], p, al)
    ser = (al * sc0) * (1.0 + t * (sc1 + t * (sc2 + t * sc3)))
    use_ser = t < ath2

    s, c = _fast_sincos(a)
    inv = 1.0 / a
    j0 = s * inv
    j1 = (j0 - c) * inv
    res = jnp.where(masks[0], j0, j1)
    jm1, jcur = j0, j1
    for i in range(2, N_SPHERICAL):
        jm1, jcur = jcur, (2 * i - 1) * inv * jcur - jm1
        res = jnp.where(masks[i], jcur, res)
    res = jnp.where(use_ser, ser, res)

    x2 = x * x
    x4 = x2 * x2
    x5 = x * x4
    env = 1.0 / x + x5 * (-28.0 + x * (48.0 + x * (-21.0)))
    env = jnp.where(x < 1.0, env, jnp.zeros_like(env))
    res = (cn * res) * env

    ct = _fast_sincos(ang_ref[pl.ds(ch * _CHUNK, _CHUNK)].reshape(1, -1))[1]
    psel = jnp.where(masks[0], jnp.ones_like(ct), ct)
    pm1, pc = jnp.ones_like(ct), ct
    for l in range(2, N_SPHERICAL):
        pm1, pc = pc, ((2 * l - 1) * ct * pc - (l - 1) * pm1) * (1.0 / l)
        psel = jnp.where(masks[l], pc, psel)
    out_ref[:, pl.ds(ch * _CHUNK, _CHUNK)] = res * psel


def _tc_compute(dist_kj, angle, block_t=8192, interpret=False):
    t = dist_kj.shape[0]
    nb = (t + block_t - 1) // block_t
    assert block_t % 1024 == 0
    const_spec = pl.BlockSpec((_NC, 1), lambda i: (0, 0))
    return pl.pallas_call(
        _tc_body,
        grid=(nb,),
        in_specs=[
            const_spec,
            const_spec,
            const_spec,
            pl.BlockSpec((_NC, 5), lambda i: (0, 0)),
            pl.BlockSpec((block_t,), lambda i: (i,)),
            pl.BlockSpec((block_t,), lambda i: (i,)),
        ],
        out_specs=pl.BlockSpec((_NC, block_t), lambda i: (0, i)),
        out_shape=jax.ShapeDtypeStruct((_NC, t), jnp.float32),
        interpret=interpret,
    )(
        jnp.asarray(_Z_FLAT)[:, None],
        jnp.asarray(_N_FLAT * _PREF_COL)[:, None],
        jnp.asarray(_L_OF_COL)[:, None],
        jnp.asarray(_SER),
        dist_kj,
        angle,
    )


def kernel(dist, angle, edge_idx_kj):
    dist_kj = _sc_gather(dist, edge_idx_kj)
    out_t = _tc_compute(dist_kj, angle)
    return jnp.transpose(out_t, (1, 0))
